# baseline (speedup 1.0000x reference)
import functools

import jax
import jax.numpy as jnp
from jax import lax
from jax.experimental import pallas as pl
from jax.experimental.pallas import tpu as pltpu
from jax.experimental.pallas import tpu_sc as plsc

_INFO = plsc.get_sparse_core_info()
_NC, _NS = _INFO.num_cores, _INFO.num_subcores
_NW = _NC * _NS


def _make_sc_copy(n_elems):
    chunk = n_elems // _NW
    mesh = plsc.VectorSubcoreMesh(core_axis_name="c", subcore_axis_name="s")

    @functools.partial(
        pl.kernel,
        mesh=mesh,
        out_type=jax.ShapeDtypeStruct((n_elems,), jnp.float32),
        scratch_types=[pltpu.SemaphoreType.DMA],
    )
    def sc_copy(in_hbm, out_hbm, sem):
        wid = lax.axis_index("s") * _NC + lax.axis_index("c")
        base = wid * chunk
        pltpu.async_copy(
            in_hbm.at[pl.ds(base, chunk)],
            out_hbm.at[pl.ds(base, chunk)],
            sem,
        ).wait()

    return sc_copy


def kernel(g, inp_h):
    n_rows, n_cols = inp_h.shape
    flat = inp_h.reshape(n_rows * n_cols)
    out = _make_sc_copy(n_rows * n_cols)(flat)
    return out.reshape(n_rows, n_cols)


# speedup vs baseline: 27.5561x; 27.5561x over previous
import functools

import jax
import jax.numpy as jnp
from jax import lax
from jax.experimental import pallas as pl
from jax.experimental.pallas import tpu as pltpu
from jax.experimental.pallas import tpu_sc as plsc

_INFO = plsc.get_sparse_core_info()
_NC, _NS = _INFO.num_cores, _INFO.num_subcores
_NW = _NC * _NS


_BUF = 16000


def _make_sc_copy(n_elems):
    chunk = n_elems // _NW
    n_iter = chunk // _BUF
    mesh = plsc.VectorSubcoreMesh(core_axis_name="c", subcore_axis_name="s")

    @functools.partial(
        pl.kernel,
        mesh=mesh,
        out_type=jax.ShapeDtypeStruct((n_elems,), jnp.float32),
        scratch_types=[
            pltpu.VMEM((2, _BUF), jnp.float32),
            pltpu.SemaphoreType.DMA((2,)),
            pltpu.SemaphoreType.DMA((2,)),
        ],
    )
    def sc_copy(in_hbm, out_hbm, bufs, in_sems, out_sems):
        wid = lax.axis_index("s") * _NC + lax.axis_index("c")
        base = wid * chunk

        def in_copy(i, slot):
            return pltpu.make_async_copy(
                in_hbm.at[pl.ds(base + i * _BUF, _BUF)],
                bufs.at[slot],
                in_sems.at[slot],
            )

        def out_copy(i, slot):
            return pltpu.make_async_copy(
                bufs.at[slot],
                out_hbm.at[pl.ds(base + i * _BUF, _BUF)],
                out_sems.at[slot],
            )

        in_copy(0, 0).start()
        for i in range(n_iter):
            slot = i % 2
            if i + 1 < n_iter:
                nxt = (i + 1) % 2
                if i >= 1:
                    out_copy(i - 1, nxt).wait()
                in_copy(i + 1, nxt).start()
            in_copy(i, slot).wait()
            out_copy(i, slot).start()
        out_copy(n_iter - 2, (n_iter - 2) % 2).wait()
        out_copy(n_iter - 1, (n_iter - 1) % 2).wait()

    return sc_copy


def kernel(g, inp_h):
    n_rows, n_cols = inp_h.shape
    flat = inp_h.reshape(n_rows * n_cols)
    out = _make_sc_copy(n_rows * n_cols)(flat)
    return out.reshape(n_rows, n_cols)
